# baseline (speedup 1.0000x reference)
import functools

import jax
import jax.numpy as jnp
from jax import lax
from jax.experimental import pallas as pl
from jax.experimental.pallas import tpu as pltpu
from jax.experimental.pallas import tpu_sc as plsc

VOCAB = 100000
EMBED = 64
N, T = 4096, 50
L = 16
UNROLL = 16

_INFO = plsc.get_sparse_core_info()
NC, NS = _INFO.num_cores, _INFO.num_subcores
NW = NC * NS
FPW = EMBED // NW
SEGS = [(0, 24), (24, 48), (48, 50)]

_mesh = plsc.VectorSubcoreMesh(core_axis_name="c", subcore_axis_name="s")


@functools.partial(
    pl.kernel,
    out_type=jax.ShapeDtypeStruct((T, EMBED, N), jnp.float32),
    mesh=_mesh,
    scratch_types=[
        pltpu.VMEM((VOCAB,), jnp.float32),
        pltpu.VMEM((2, N), jnp.int32),
        pltpu.VMEM((2, 2, N), jnp.float32),
        pltpu.VMEM_SHARED((3, 8, N), jnp.int32),
        pltpu.SemaphoreType.DMA,
        pltpu.SemaphoreType.DMA,
        pltpu.SemaphoreType.DMA,
        pltpu.SemaphoreType.DMA,
    ],
    compiler_params=pltpu.CompilerParams(
        use_tc_tiling_on_sc=True, needs_layout_passes=False
    ),
)
def _embed_lookup(xt_hbm, wt_hbm, out_hbm, wrow, xrow, orow, xsh, x0, x1, o0, o1):
    cid = lax.axis_index("c")
    sid = lax.axis_index("s")
    xsem = [x0, x1]
    osem = [o0, o1]

    def xsh_row(tb):
        return xsh.at[lax.rem(lax.div(tb, 8), 3), lax.rem(tb, 8)]

    def gather_row(pb, b):
        @plsc.parallel_loop(0, N, L, unroll=UNROLL)
        def _(i):
            idx16 = xrow[b, pl.ds(i, L)]
            orow[pb, b, pl.ds(i, L)] = plsc.load_gather(wrow, [idx16])

    for f in range(FPW):
        d = cid * NS + NW * f + sid
        pltpu.sync_copy(wt_hbm.at[d], wrow)

        for a, bnd in SEGS:
            plsc.subcore_barrier()

            @pl.when(sid == 0)
            def _(a=a, bnd=bnd):
                for k in range((bnd - a + 7) // 8):
                    r0 = a + 8 * k
                    rows = min(8, bnd - r0)
                    if rows == 8:
                        pltpu.sync_copy(
                            xt_hbm.at[pl.ds(r0, 8)], xsh.at[(r0 // 8) % 3]
                        )
                    else:
                        pltpu.sync_copy(
                            xt_hbm.at[pl.ds(r0, rows)],
                            xsh.at[(r0 // 8) % 3, pl.ds(0, rows)],
                        )

            plsc.subcore_barrier()

            pltpu.async_copy(xsh_row(a), xrow.at[0], xsem[0])
            if a + 1 < bnd:
                pltpu.async_copy(xsh_row(a + 1), xrow.at[1], xsem[1])

            def do_pair(tt, pb, bnd=bnd):
                @pl.when(tt >= 4)
                def _():
                    pltpu.make_async_copy(
                        orow.at[pb], out_hbm.at[pl.ds(tt - 4, 2), d], osem[pb]
                    ).wait()

                for b in range(2):
                    tb = tt + b
                    pltpu.make_async_copy(
                        xsh_row(tb), xrow.at[b], xsem[b]
                    ).wait()
                    gather_row(pb, b)

                    @pl.when(tb + 2 < bnd)
                    def _():
                        pltpu.async_copy(xsh_row(tb + 2), xrow.at[b], xsem[b])

                pltpu.async_copy(
                    orow.at[pb], out_hbm.at[pl.ds(tt, 2), d], osem[pb]
                )

            npairs = (bnd - a) // 2
            if npairs >= 2:

                def quad(q, carry, a=a):
                    do_pair(a + 4 * q, 0)
                    do_pair(a + 4 * q + 2, 1)
                    return carry

                lax.fori_loop(0, npairs // 2, quad, 0)
            else:
                do_pair(a, 0)

        pltpu.make_async_copy(
            orow.at[1], out_hbm.at[pl.ds(T - 4, 2), d], osem[1]
        ).wait()
        pltpu.make_async_copy(
            orow.at[0], out_hbm.at[pl.ds(T - 2, 2), d], osem[0]
        ).wait()


def kernel(x, W_embed):
    out_t = _embed_lookup(x.T, W_embed.T)
    return out_t.transpose(2, 0, 1)


# speedup vs baseline: 1.1445x; 1.1445x over previous
import functools

import jax
import jax.numpy as jnp
from jax import lax
from jax.experimental import pallas as pl
from jax.experimental.pallas import tpu as pltpu
from jax.experimental.pallas import tpu_sc as plsc

VOCAB = 100000
EMBED = 64
N, T = 4096, 50
L = 16
UNROLL = 16

_INFO = plsc.get_sparse_core_info()
NC, NS = _INFO.num_cores, _INFO.num_subcores
NW = NC * NS
FPW = EMBED // NW

_mesh = plsc.VectorSubcoreMesh(core_axis_name="c", subcore_axis_name="s")


@functools.partial(
    pl.kernel,
    out_type=jax.ShapeDtypeStruct((T, EMBED, N), jnp.float32),
    mesh=_mesh,
    scratch_types=[
        pltpu.VMEM((VOCAB,), jnp.float32),
        pltpu.VMEM((2, N), jnp.int32),
        pltpu.VMEM((2, N), jnp.float32),
        pltpu.VMEM_SHARED((7, 8, N), jnp.int32),
        pltpu.SemaphoreType.DMA,
        pltpu.SemaphoreType.DMA,
        pltpu.SemaphoreType.DMA,
        pltpu.SemaphoreType.DMA,
    ],
    compiler_params=pltpu.CompilerParams(
        use_tc_tiling_on_sc=True, needs_layout_passes=False
    ),
)
def _embed_lookup(xt_hbm, wt_hbm, out_hbm, wrow, xrow, orow, xsh, x0, x1, o0, o1):
    cid = lax.axis_index("c")
    sid = lax.axis_index("s")
    xsem = [x0, x1]
    osem = [o0, o1]

    for k in range(T // 8):
        @pl.when(sid == k)
        def _(k=k):
            pltpu.sync_copy(xt_hbm.at[pl.ds(8 * k, 8)], xsh.at[k])

    @pl.when(sid == T // 8)
    def _():
        pltpu.sync_copy(
            xt_hbm.at[pl.ds(8 * (T // 8), T % 8)],
            xsh.at[T // 8, pl.ds(0, T % 8)],
        )

    plsc.subcore_barrier()

    def xsh_row(tb):
        return xsh.at[lax.div(tb, 8), lax.rem(tb, 8)]

    def gather_row(b):
        @plsc.parallel_loop(0, N, L, unroll=UNROLL)
        def _(i):
            idx16 = xrow[b, pl.ds(i, L)]
            orow[b, pl.ds(i, L)] = plsc.load_gather(wrow, [idx16])

    for f in range(FPW):
        d = cid * NS + NW * f + sid
        pltpu.sync_copy(wt_hbm.at[d], wrow)

        pltpu.async_copy(xsh.at[0, 0], xrow.at[0], xsem[0])
        pltpu.async_copy(xsh.at[0, 1], xrow.at[1], xsem[1])

        def tpair(i, carry):
            t = 2 * i
            for b in range(2):
                tb = t + b

                pltpu.make_async_copy(
                    xsh_row(tb), xrow.at[b], xsem[b]
                ).wait()

                @pl.when(tb >= 2)
                def _():
                    pltpu.make_async_copy(
                        orow.at[b], out_hbm.at[tb - 2, d], osem[b]
                    ).wait()

                gather_row(b)

                pltpu.async_copy(orow.at[b], out_hbm.at[tb, d], osem[b])

                @pl.when(tb + 2 < T)
                def _():
                    pltpu.async_copy(xsh_row(tb + 2), xrow.at[b], xsem[b])

            return carry

        lax.fori_loop(0, T // 2, tpair, 0)

        pltpu.make_async_copy(orow.at[0], out_hbm.at[T - 2, d], osem[0]).wait()
        pltpu.make_async_copy(orow.at[1], out_hbm.at[T - 1, d], osem[1]).wait()


def kernel(x, W_embed):
    out_t = _embed_lookup(x.T, W_embed.T)
    return out_t.transpose(2, 0, 1)


# speedup vs baseline: 1.1454x; 1.0008x over previous
import functools

import jax
import jax.numpy as jnp
from jax import lax
from jax.experimental import pallas as pl
from jax.experimental.pallas import tpu as pltpu
from jax.experimental.pallas import tpu_sc as plsc

VOCAB = 100000
EMBED = 64
N, T = 4096, 50
L = 16
UNROLL = 32

_INFO = plsc.get_sparse_core_info()
NC, NS = _INFO.num_cores, _INFO.num_subcores
NW = NC * NS
FPW = EMBED // NW

_mesh = plsc.VectorSubcoreMesh(core_axis_name="c", subcore_axis_name="s")


@functools.partial(
    pl.kernel,
    out_type=jax.ShapeDtypeStruct((T, EMBED, N), jnp.float32),
    mesh=_mesh,
    scratch_types=[
        pltpu.VMEM((VOCAB,), jnp.float32),
        pltpu.VMEM((2, N), jnp.int32),
        pltpu.VMEM((2, N), jnp.float32),
        pltpu.VMEM_SHARED((7, 8, N), jnp.int32),
        pltpu.SemaphoreType.DMA,
        pltpu.SemaphoreType.DMA,
        pltpu.SemaphoreType.DMA,
        pltpu.SemaphoreType.DMA,
    ],
    compiler_params=pltpu.CompilerParams(
        use_tc_tiling_on_sc=True, needs_layout_passes=False
    ),
)
def _embed_lookup(xt_hbm, wt_hbm, out_hbm, wrow, xrow, orow, xsh, x0, x1, o0, o1):
    cid = lax.axis_index("c")
    sid = lax.axis_index("s")
    xsem = [x0, x1]
    osem = [o0, o1]

    for k in range(T // 8):
        @pl.when(sid == k)
        def _(k=k):
            pltpu.sync_copy(xt_hbm.at[pl.ds(8 * k, 8)], xsh.at[k])

    @pl.when(sid == T // 8)
    def _():
        pltpu.sync_copy(
            xt_hbm.at[pl.ds(8 * (T // 8), T % 8)],
            xsh.at[T // 8, pl.ds(0, T % 8)],
        )

    plsc.subcore_barrier()

    def xsh_row(tb):
        return xsh.at[lax.div(tb, 8), lax.rem(tb, 8)]

    def gather_row(b):
        @plsc.parallel_loop(0, N, L, unroll=UNROLL)
        def _(i):
            idx16 = xrow[b, pl.ds(i, L)]
            orow[b, pl.ds(i, L)] = plsc.load_gather(wrow, [idx16])

    for f in range(FPW):
        d = cid * NS + NW * f + sid
        pltpu.sync_copy(wt_hbm.at[d], wrow)

        pltpu.async_copy(xsh.at[0, 0], xrow.at[0], xsem[0])
        pltpu.async_copy(xsh.at[0, 1], xrow.at[1], xsem[1])

        def tpair(i, carry):
            t = 2 * i
            for b in range(2):
                tb = t + b

                pltpu.make_async_copy(
                    xsh_row(tb), xrow.at[b], xsem[b]
                ).wait()

                @pl.when(tb >= 2)
                def _():
                    pltpu.make_async_copy(
                        orow.at[b], out_hbm.at[tb - 2, d], osem[b]
                    ).wait()

                gather_row(b)

                pltpu.async_copy(orow.at[b], out_hbm.at[tb, d], osem[b])

                @pl.when(tb + 2 < T)
                def _():
                    pltpu.async_copy(xsh_row(tb + 2), xrow.at[b], xsem[b])

            return carry

        lax.fori_loop(0, T // 2, tpair, 0)

        pltpu.make_async_copy(orow.at[0], out_hbm.at[T - 2, d], osem[0]).wait()
        pltpu.make_async_copy(orow.at[1], out_hbm.at[T - 1, d], osem[1]).wait()


def kernel(x, W_embed):
    out_t = _embed_lookup(x.T, W_embed.T)
    return out_t.transpose(2, 0, 1)
